# baseline (speedup 1.0000x reference)
import functools

import jax
import jax.numpy as jnp
from jax import lax
from jax.experimental import pallas as pl
from jax.experimental.pallas import tpu as pltpu
from jax.experimental.pallas import tpu_sc as plsc

_B, _S = 4, 4096
_N = _B * _S
_NW = 32
_CH = _N // _NW
_LANES = 16


def _threefry_bits(seed, idx_u32):
    ks0 = jnp.uint32(0)
    ks1 = jnp.uint32(seed)
    ks2 = ks0 ^ ks1 ^ jnp.uint32(0x1BD11BDA)
    ks = (ks0, ks1, ks2)
    rotations = ((13, 15, 26, 6), (17, 29, 16, 24))
    x0 = jnp.zeros((_LANES,), jnp.uint32) + ks0
    x1 = idx_u32 + ks1
    for i in range(5):
        for r in rotations[i % 2]:
            x0 = x0 + x1
            x1 = (x1 << jnp.uint32(r)) | (x1 >> jnp.uint32(32 - r))
            x1 = x0 ^ x1
        x0 = x0 + ks[(i + 1) % 3]
        x1 = x1 + ks[(i + 2) % 3] + jnp.uint32(i + 1)
    return x0 ^ x1


def _to_uniform(bits):
    fb = (bits >> jnp.uint32(9)) | jnp.uint32(0x3F800000)
    return lax.bitcast_convert_type(fb, jnp.float32) - jnp.float32(1.0)


def _uniform_at(seed, idx_i32):
    return _to_uniform(
        _threefry_bits(seed, lax.bitcast_convert_type(idx_i32, jnp.uint32)))


def _step(j_base, lane, xv):
    j = j_base + lane
    l0 = _uniform_at(1, j * 2)
    l1 = _uniform_at(1, j * 2 + 1)
    um = _uniform_at(2, j)
    m = jnp.maximum(l0, l1)
    e0 = jnp.exp(l0 - m)
    e1 = jnp.exp(l1 - m)
    s = e0 + e1
    p0 = e0 / s
    p1 = e1 / s
    mk = um < jnp.float32(0.5)
    mask01 = jnp.where(mk, jnp.full((_LANES,), 1, jnp.int32),
                       jnp.full((_LANES,), 0, jnp.int32))
    chosen = jnp.where(mk, p1, p0)
    masked = jnp.where(mk, xv, jnp.zeros((_LANES,), jnp.float32))
    return l0, l1, p0, p1, chosen, mask01, masked


@functools.cache
def _build_rre_kernel():
    mesh = plsc.VectorSubcoreMesh(core_axis_name="c", subcore_axis_name="s")

    @functools.partial(
        pl.kernel,
        mesh=mesh,
        out_type=[
            jax.ShapeDtypeStruct((_N,), jnp.float32),
            jax.ShapeDtypeStruct((_N,), jnp.float32),
            jax.ShapeDtypeStruct((_N,), jnp.float32),
            jax.ShapeDtypeStruct((_N,), jnp.float32),
            jax.ShapeDtypeStruct((_N,), jnp.float32),
            jax.ShapeDtypeStruct((_N,), jnp.int32),
            jax.ShapeDtypeStruct((_N,), jnp.float32),
        ],
        scratch_types=[
            pltpu.VMEM((_CH,), jnp.float32),
            pltpu.VMEM((_CH,), jnp.float32),
            pltpu.VMEM((_CH,), jnp.float32),
            pltpu.VMEM((_CH,), jnp.float32),
            pltpu.VMEM((_CH,), jnp.float32),
            pltpu.VMEM((_CH,), jnp.float32),
            pltpu.VMEM((_CH,), jnp.int32),
            pltpu.VMEM((_CH,), jnp.float32),
        ],
    )
    def _rre_kernel(x_hbm, l0_hbm, l1_hbm, p0_hbm, p1_hbm, ch_hbm, m_hbm,
                    mi_hbm, x_v, l0_v, l1_v, p0_v, p1_v, ch_v, m_v, mi_v):
        wid = lax.axis_index("s") * 2 + lax.axis_index("c")
        jb = wid * _CH
        pltpu.sync_copy(x_hbm.at[pl.ds(jb, _CH)], x_v)

        def body(i, carry):
            lane = lax.iota(jnp.int32, _LANES)
            o = i * _LANES
            xv = x_v[pl.ds(o, _LANES)]
            l0, l1, p0, p1, ch, m01, masked = _step(jb + o, lane, xv)
            l0_v[pl.ds(o, _LANES)] = l0
            l1_v[pl.ds(o, _LANES)] = l1
            p0_v[pl.ds(o, _LANES)] = p0
            p1_v[pl.ds(o, _LANES)] = p1
            ch_v[pl.ds(o, _LANES)] = ch
            m_v[pl.ds(o, _LANES)] = m01
            mi_v[pl.ds(o, _LANES)] = masked
            return carry

        lax.fori_loop(0, _CH // _LANES, body, 0)

        pltpu.sync_copy(l0_v, l0_hbm.at[pl.ds(jb, _CH)])
        pltpu.sync_copy(l1_v, l1_hbm.at[pl.ds(jb, _CH)])
        pltpu.sync_copy(p0_v, p0_hbm.at[pl.ds(jb, _CH)])
        pltpu.sync_copy(p1_v, p1_hbm.at[pl.ds(jb, _CH)])
        pltpu.sync_copy(ch_v, ch_hbm.at[pl.ds(jb, _CH)])
        pltpu.sync_copy(m_v, m_hbm.at[pl.ds(jb, _CH)])
        pltpu.sync_copy(mi_v, mi_hbm.at[pl.ds(jb, _CH)])

    return _rre_kernel


def kernel(x):
    B, S = x.shape
    l0, l1, p0, p1, ch, m01, mi = _build_rre_kernel()(x.reshape(-1))
    logits = jnp.stack([l0, l1], axis=-1).reshape(B, S, 2)
    policy = jnp.stack([p0, p1], axis=-1).reshape(B, S, 2)
    chosen = ch.reshape(B, S, 1)
    mask = m01.astype(bool).reshape(B, S)
    masked = mi.reshape(B, S)
    return logits, policy, chosen, mask, masked


# speedup vs baseline: 1.0128x; 1.0128x over previous
import functools

import jax
import jax.numpy as jnp
from jax import lax
from jax.experimental import pallas as pl
from jax.experimental.pallas import tpu as pltpu
from jax.experimental.pallas import tpu_sc as plsc

_B, _S = 4, 4096
_N = _B * _S
_NW = 32
_CH = _N // _NW
_LANES = 16


def _threefry_bits(seed, idx_u32):
    ks0 = jnp.uint32(0)
    ks1 = jnp.uint32(seed)
    ks2 = ks0 ^ ks1 ^ jnp.uint32(0x1BD11BDA)
    ks = (ks0, ks1, ks2)
    rotations = ((13, 15, 26, 6), (17, 29, 16, 24))
    x0 = jnp.zeros((_LANES,), jnp.uint32) + ks0
    x1 = idx_u32 + ks1
    for i in range(5):
        for r in rotations[i % 2]:
            x0 = x0 + x1
            x1 = (x1 << jnp.uint32(r)) | (x1 >> jnp.uint32(32 - r))
            x1 = x0 ^ x1
        x0 = x0 + ks[(i + 1) % 3]
        x1 = x1 + ks[(i + 2) % 3] + jnp.uint32(i + 1)
    return x0 ^ x1


def _to_uniform(bits):
    fb = (bits >> jnp.uint32(9)) | jnp.uint32(0x3F800000)
    return lax.bitcast_convert_type(fb, jnp.float32) - jnp.float32(1.0)


def _uniform_at(seed, idx_i32):
    return _to_uniform(
        _threefry_bits(seed, lax.bitcast_convert_type(idx_i32, jnp.uint32)))


def _step(j_base, lane, xv):
    j = j_base + lane
    l0 = _uniform_at(1, j * 2)
    l1 = _uniform_at(1, j * 2 + 1)
    um = _uniform_at(2, j)
    m = jnp.maximum(l0, l1)
    e0 = jnp.exp(l0 - m)
    e1 = jnp.exp(l1 - m)
    s = e0 + e1
    p0 = e0 / s
    p1 = e1 / s
    mk = um < jnp.float32(0.5)
    mask01 = jnp.where(mk, jnp.full((_LANES,), 1, jnp.int32),
                       jnp.full((_LANES,), 0, jnp.int32))
    chosen = jnp.where(mk, p1, p0)
    masked = jnp.where(mk, xv, jnp.zeros((_LANES,), jnp.float32))
    return l0, l1, p0, p1, chosen, mask01, masked


@functools.cache
def _build_rre_kernel():
    mesh = plsc.VectorSubcoreMesh(core_axis_name="c", subcore_axis_name="s")

    @functools.partial(
        pl.kernel,
        mesh=mesh,
        out_type=[
            jax.ShapeDtypeStruct((_N,), jnp.float32),
            jax.ShapeDtypeStruct((_N,), jnp.float32),
            jax.ShapeDtypeStruct((_N,), jnp.float32),
            jax.ShapeDtypeStruct((_N,), jnp.float32),
            jax.ShapeDtypeStruct((_N,), jnp.float32),
            jax.ShapeDtypeStruct((_N,), jnp.int32),
            jax.ShapeDtypeStruct((_N,), jnp.float32),
        ],
        scratch_types=[
            pltpu.VMEM((_CH,), jnp.float32),
            pltpu.VMEM((_CH,), jnp.float32),
            pltpu.VMEM((_CH,), jnp.float32),
            pltpu.VMEM((_CH,), jnp.float32),
            pltpu.VMEM((_CH,), jnp.float32),
            pltpu.VMEM((_CH,), jnp.float32),
            pltpu.VMEM((_CH,), jnp.int32),
            pltpu.VMEM((_CH,), jnp.float32),
            pltpu.SemaphoreType.DMA,
        ],
    )
    def _rre_kernel(x_hbm, l0_hbm, l1_hbm, p0_hbm, p1_hbm, ch_hbm, m_hbm,
                    mi_hbm, x_v, l0_v, l1_v, p0_v, p1_v, ch_v, m_v, mi_v,
                    sem):
        wid = lax.axis_index("s") * 2 + lax.axis_index("c")
        jb = wid * _CH
        pltpu.sync_copy(x_hbm.at[pl.ds(jb, _CH)], x_v)

        def body(i, carry):
            lane = lax.iota(jnp.int32, _LANES)
            o = i * _LANES
            xv = x_v[pl.ds(o, _LANES)]
            l0, l1, p0, p1, ch, m01, masked = _step(jb + o, lane, xv)
            l0_v[pl.ds(o, _LANES)] = l0
            l1_v[pl.ds(o, _LANES)] = l1
            p0_v[pl.ds(o, _LANES)] = p0
            p1_v[pl.ds(o, _LANES)] = p1
            ch_v[pl.ds(o, _LANES)] = ch
            m_v[pl.ds(o, _LANES)] = m01
            mi_v[pl.ds(o, _LANES)] = masked
            return carry

        lax.fori_loop(0, _CH // _LANES, body, 0)

        sl = pl.ds(jb, _CH)
        copies = [
            pltpu.async_copy(l0_v, l0_hbm.at[sl], sem),
            pltpu.async_copy(l1_v, l1_hbm.at[sl], sem),
            pltpu.async_copy(p0_v, p0_hbm.at[sl], sem),
            pltpu.async_copy(p1_v, p1_hbm.at[sl], sem),
            pltpu.async_copy(ch_v, ch_hbm.at[sl], sem),
            pltpu.async_copy(m_v, m_hbm.at[sl], sem),
            pltpu.async_copy(mi_v, mi_hbm.at[sl], sem),
        ]
        for c in copies:
            c.wait()

    return _rre_kernel


def kernel(x):
    B, S = x.shape
    l0, l1, p0, p1, ch, m01, mi = _build_rre_kernel()(x.reshape(-1))
    logits = jnp.stack([l0, l1], axis=-1).reshape(B, S, 2)
    policy = jnp.stack([p0, p1], axis=-1).reshape(B, S, 2)
    chosen = ch.reshape(B, S, 1)
    mask = m01.astype(bool).reshape(B, S)
    masked = mi.reshape(B, S)
    return logits, policy, chosen, mask, masked
